# Initial kernel scaffold; baseline (speedup 1.0000x reference)
import jax
import jax.numpy as jnp
from jax.experimental import pallas as pl


def kernel(inputs, weight):
    raise NotImplementedError("write your pallas kernel here")


# speedup vs baseline: 2.4736x; 2.4736x over previous
import functools

import jax
import jax.numpy as jnp
from jax import lax
from jax.experimental import pallas as pl
from jax.experimental.pallas import tpu as pltpu
from jax.experimental.pallas import tpu_sc as plsc

_K = 8192
_D = 256
_N = 8192
_TM = 256
_COMMITMENT_COST = 0.25


def _wnorm_body(w_ref, wn_ref):
    w = w_ref[...]
    n = jnp.sqrt(jnp.sum(w * w, axis=-1, keepdims=True))
    wn_ref[...] = w / jnp.clip(n, 1e-12, None)


def _normalize_weight(w):
    return pl.pallas_call(
        _wnorm_body,
        out_shape=jax.ShapeDtypeStruct(w.shape, w.dtype),
    )(w)


def _argmax_onehot_body(x_ref, wn_ref, idx_ref, oh_ref):
    x = x_ref[...]
    n = jnp.sqrt(jnp.sum(x * x, axis=-1, keepdims=True))
    xn = x / jnp.clip(n, 1e-12, None)
    wn = wn_ref[...]
    dist = lax.dot_general(
        xn, wn, (((1,), (1,)), ((), ())),
        preferred_element_type=jnp.float32)
    kdim = dist.shape[1]
    m = jnp.max(dist, axis=1, keepdims=True)
    col = lax.broadcasted_iota(jnp.int32, dist.shape, 1)
    idx = jnp.min(jnp.where(dist == m, col, kdim), axis=1)
    idx_ref[...] = idx.reshape(1, 1, idx.shape[0])
    oh_ref[...] = (col == idx[:, None]).astype(jnp.float32)


def _argmax_onehot(x_flat, wn):
    n, d = x_flat.shape
    k = wn.shape[0]
    g = n // _TM
    idx3, onehot = pl.pallas_call(
        _argmax_onehot_body,
        grid=(g,),
        in_specs=[
            pl.BlockSpec((_TM, d), lambda i: (i, 0)),
            pl.BlockSpec((k, d), lambda i: (0, 0)),
        ],
        out_specs=[
            pl.BlockSpec((1, 1, _TM), lambda i: (i, 0, 0)),
            pl.BlockSpec((_TM, k), lambda i: (i, 0)),
        ],
        out_shape=[
            jax.ShapeDtypeStruct((g, 1, _TM), jnp.int32),
            jax.ShapeDtypeStruct((n, k), jnp.float32),
        ],
    )(x_flat, wn)
    return idx3.reshape(n), onehot


@functools.lru_cache(maxsize=None)
def _make_sc_gather(n, k, d):
    info = plsc.get_sparse_core_info()
    nw = info.num_cores * info.num_subcores
    bpw = n // nw
    mesh = plsc.VectorSubcoreMesh(core_axis_name="c", subcore_axis_name="s")

    @functools.partial(
        pl.kernel,
        out_type=jax.ShapeDtypeStruct((n, d), jnp.float32),
        mesh=mesh,
        scratch_types=[
            pltpu.VMEM((bpw,), jnp.int32),
            pltpu.VMEM((bpw, d), jnp.float32),
            pltpu.SemaphoreType.DMA,
        ],
    )
    def gather_kernel(table_hbm, idx_hbm, out_hbm, idx_v, rows_v, sem):
        wid = lax.axis_index("s") * info.num_cores + lax.axis_index("c")
        base = wid * bpw
        pltpu.sync_copy(idx_hbm.at[pl.ds(base, bpw)], idx_v)
        pltpu.async_copy(table_hbm.at[idx_v], rows_v, sem).wait()
        pltpu.sync_copy(rows_v, out_hbm.at[pl.ds(base, bpw)])

    return gather_kernel


def _loss_body(x_ref, q_ref, idx_ref, st_ref, sc_ref):
    x = x_ref[...]
    q = q_ref[...]
    st_ref[...] = x + (q - x)
    diff = q - x
    m = jnp.sum(diff * diff) * (1.0 / (_N * _D))
    idx = idx_ref[...]
    eq = (idx[:, None, :] == idx[None, :, :]).astype(jnp.float32)
    c = jnp.sum(eq, axis=0)
    s = jnp.sum(jnp.log(c * 0.125 + 1e-10)) * 0.125
    sc_ref[0] = m
    sc_ref[1] = _COMMITMENT_COST * m
    sc_ref[2] = jnp.exp(-s)


def _losses(x_flat, quant, idx_bt):
    return pl.pallas_call(
        _loss_body,
        out_specs=[
            pl.BlockSpec(memory_space=pltpu.MemorySpace.VMEM),
            pl.BlockSpec(memory_space=pltpu.MemorySpace.SMEM),
        ],
        out_shape=[
            jax.ShapeDtypeStruct(x_flat.shape, jnp.float32),
            jax.ShapeDtypeStruct((4,), jnp.float32),
        ],
    )(x_flat, quant, idx_bt)


def kernel(inputs, weight):
    b, t, d = inputs.shape
    k = weight.shape[0]
    x_flat = inputs.reshape(b * t, d)
    wn = _normalize_weight(weight)
    idx_flat, onehot = _argmax_onehot(x_flat, wn)
    quant = _make_sc_gather(b * t, k, d)(weight, idx_flat)
    st_flat, scalars = _losses(x_flat, quant, idx_flat.reshape(b, t))
    quantized_st = st_flat.reshape(b, t, d)
    encoding_indices_out = idx_flat.reshape(b, t, 1)
    min_encodings = onehot.reshape(b, t, k)
    return (quantized_st, encoding_indices_out, scalars[0], scalars[1],
            scalars[2], min_encodings)


# speedup vs baseline: 2.7723x; 1.1208x over previous
import functools

import jax
import jax.numpy as jnp
from jax import lax
from jax.experimental import pallas as pl
from jax.experimental.pallas import tpu as pltpu
from jax.experimental.pallas import tpu_sc as plsc

_K = 8192
_D = 256
_N = 8192
_TM = 256
_COMMITMENT_COST = 0.25


def _argmax_onehot_body(x_ref, w_ref, idx_ref, oh_ref, wn_ref):
    @pl.when(pl.program_id(0) == 0)
    def _():
        w = w_ref[...]
        n = jnp.sqrt(jnp.sum(w * w, axis=-1, keepdims=True))
        wn_ref[...] = w / jnp.clip(n, 1e-12, None)

    x = x_ref[...]
    n = jnp.sqrt(jnp.sum(x * x, axis=-1, keepdims=True))
    xn = x / jnp.clip(n, 1e-12, None)
    dist = lax.dot_general(
        xn, wn_ref[...], (((1,), (1,)), ((), ())),
        preferred_element_type=jnp.float32)
    kdim = dist.shape[1]
    m = jnp.max(dist, axis=1, keepdims=True)
    colf = lax.broadcasted_iota(jnp.int32, dist.shape, 1).astype(jnp.float32)
    vf = jnp.where(dist == m, colf, float(kdim))
    idxf = jnp.min(vf, axis=1, keepdims=True)
    idx_ref[...] = idxf.astype(jnp.int32).reshape(1, 1, idxf.shape[0])
    oh_ref[...] = (vf == idxf).astype(jnp.float32)


def _argmax_onehot(x_flat, w):
    n, d = x_flat.shape
    k = w.shape[0]
    g = n // _TM
    idx3, onehot = pl.pallas_call(
        _argmax_onehot_body,
        grid=(g,),
        in_specs=[
            pl.BlockSpec((_TM, d), lambda i: (i, 0)),
            pl.BlockSpec((k, d), lambda i: (0, 0)),
        ],
        out_specs=[
            pl.BlockSpec((1, 1, _TM), lambda i: (i, 0, 0)),
            pl.BlockSpec((_TM, k), lambda i: (i, 0)),
        ],
        out_shape=[
            jax.ShapeDtypeStruct((g, 1, _TM), jnp.int32),
            jax.ShapeDtypeStruct((n, k), jnp.float32),
        ],
        scratch_shapes=[pltpu.VMEM((k, d), jnp.float32)],
    )(x_flat, w)
    return idx3.reshape(n), onehot


@functools.lru_cache(maxsize=None)
def _make_sc_gather(n, d):
    info = plsc.get_sparse_core_info()
    nw = info.num_cores * info.num_subcores
    bpw = n // nw
    mesh = plsc.VectorSubcoreMesh(core_axis_name="c", subcore_axis_name="s")

    @functools.partial(
        pl.kernel,
        out_type=jax.ShapeDtypeStruct((n, d), jnp.float32),
        mesh=mesh,
        scratch_types=[
            pltpu.VMEM((bpw,), jnp.int32),
            pltpu.VMEM((bpw, d), jnp.float32),
            pltpu.SemaphoreType.DMA,
        ],
    )
    def gather_kernel(table_hbm, idx_hbm, out_hbm, idx_v, rows_v, sem):
        wid = lax.axis_index("s") * info.num_cores + lax.axis_index("c")
        base = wid * bpw
        pltpu.sync_copy(idx_hbm.at[pl.ds(base, bpw)], idx_v)
        pltpu.async_copy(table_hbm.at[idx_v], rows_v, sem).wait()
        pltpu.sync_copy(rows_v, out_hbm.at[pl.ds(base, bpw)])

    return gather_kernel


def _loss_body(x_ref, q_ref, idx_ref, sc_ref):
    diff = q_ref[...] - x_ref[...]
    m = jnp.sum(diff * diff) * (1.0 / (_N * _D))
    idx = idx_ref[...]
    eq = (idx[:, None, :] == idx[None, :, :]).astype(jnp.float32)
    c = jnp.sum(eq, axis=0)
    s = jnp.sum(jnp.log(c * 0.125 + 1e-10)) * 0.125
    sc_ref[0] = m
    sc_ref[1] = _COMMITMENT_COST * m
    sc_ref[2] = jnp.exp(-s)


def _losses(x_flat, quant, idx_bt):
    return pl.pallas_call(
        _loss_body,
        out_specs=pl.BlockSpec(memory_space=pltpu.MemorySpace.SMEM),
        out_shape=jax.ShapeDtypeStruct((4,), jnp.float32),
    )(x_flat, quant, idx_bt)


def kernel(inputs, weight):
    b, t, d = inputs.shape
    k = weight.shape[0]
    x_flat = inputs.reshape(b * t, d)
    idx_flat, onehot = _argmax_onehot(x_flat, weight)
    quant = _make_sc_gather(b * t, d)(weight, idx_flat)
    scalars = _losses(x_flat, quant, idx_flat.reshape(b, t))
    quantized_st = quant.reshape(b, t, d)
    encoding_indices_out = idx_flat.reshape(b, t, 1)
    min_encodings = onehot.reshape(b, t, k)
    return (quantized_st, encoding_indices_out, scalars[0], scalars[1],
            scalars[2], min_encodings)
